# baseline (speedup 1.0000x reference)
import functools

import jax
import jax.numpy as jnp
from jax import lax
from jax.experimental import pallas as pl
from jax.experimental.pallas import tpu as pltpu
from jax.experimental.pallas import tpu_sc as plsc

E = 8
K = 2
D = 2048
F = 2048
T = 2048
NTOT = T * K

BM = 128
NP = NTOT + E * BM
NB = NP // BM
BF = 1024
NF1 = F // BF
NF2 = D // BF

NC, NS = 2, 16
NW = NC * NS
CH = 32

BT = 256


def _sc_gather(table, idx, n_rows):
    rows_per_w = n_rows // NW
    chunks = rows_per_w // CH
    mesh = plsc.VectorSubcoreMesh(core_axis_name="c", subcore_axis_name="s")

    @functools.partial(
        pl.kernel,
        mesh=mesh,
        out_type=jax.ShapeDtypeStruct((n_rows, D), jnp.float32),
        scratch_types=[
            pltpu.VMEM((CH,), jnp.int32),
            pltpu.VMEM((CH, D), jnp.float32),
            pltpu.SemaphoreType.DMA,
        ],
    )
    def gk(table_hbm, idx_hbm, out_hbm, idx_v, rows_v, sem):
        wid = lax.axis_index("s") * NC + lax.axis_index("c")
        for t in range(chunks):
            base = (wid * chunks + t) * CH
            pltpu.sync_copy(idx_hbm.at[pl.ds(base, CH)], idx_v)
            pltpu.async_copy(table_hbm.at[idx_v], rows_v, sem).wait()
            pltpu.sync_copy(rows_v, out_hbm.at[pl.ds(base, CH)])

    return gk(table, idx)


def _stage1(be, x_pad, gate_up):
    def body(be_ref, x_ref, wg_ref, wu_ref, h_ref):
        del be_ref
        x = x_ref[...].astype(jnp.bfloat16)
        g = jnp.dot(x, wg_ref[0], preferred_element_type=jnp.float32)
        u = jnp.dot(x, wu_ref[0], preferred_element_type=jnp.float32)
        h_ref[...] = ((g * jax.nn.sigmoid(g)) * u).astype(jnp.bfloat16)

    grid_spec = pltpu.PrefetchScalarGridSpec(
        num_scalar_prefetch=1,
        grid=(NF1, NB),
        in_specs=[
            pl.BlockSpec((BM, D), lambda f, r, be: (r, 0)),
            pl.BlockSpec((1, D, BF), lambda f, r, be: (be[r], 0, f)),
            pl.BlockSpec((1, D, BF), lambda f, r, be: (be[r], 0, f + NF1)),
        ],
        out_specs=pl.BlockSpec((BM, BF), lambda f, r, be: (r, f)),
    )
    return pl.pallas_call(
        body,
        grid_spec=grid_spec,
        out_shape=jax.ShapeDtypeStruct((NP, F), jnp.bfloat16),
    )(be, x_pad, gate_up, gate_up)


def _stage2(be, h_pad, down_proj):
    def body(be_ref, h_ref, wd_ref, y_ref):
        del be_ref
        y_ref[...] = jnp.dot(h_ref[...], wd_ref[0],
                             preferred_element_type=jnp.float32)

    grid_spec = pltpu.PrefetchScalarGridSpec(
        num_scalar_prefetch=1,
        grid=(NF2, NB),
        in_specs=[
            pl.BlockSpec((BM, F), lambda f, r, be: (r, 0)),
            pl.BlockSpec((1, F, BF), lambda f, r, be: (be[r], 0, f)),
        ],
        out_specs=pl.BlockSpec((BM, BF), lambda f, r, be: (r, f)),
    )
    return pl.pallas_call(
        body,
        grid_spec=grid_spec,
        out_shape=jax.ShapeDtypeStruct((NP, D), jnp.float32),
    )(be, h_pad, down_proj)


def _combine(y01, w0b, w1b):
    nb = T // BT

    def body(ya_ref, yb_ref, wa_ref, wb_ref, o_ref):
        o_ref[...] = (wa_ref[:, :1] * ya_ref[...]
                      + wb_ref[:, :1] * yb_ref[...])

    return pl.pallas_call(
        body,
        grid=(nb,),
        in_specs=[
            pl.BlockSpec((BT, D), lambda r: (r, 0)),
            pl.BlockSpec((BT, D), lambda r: (r + nb, 0)),
            pl.BlockSpec((BT, 128), lambda r: (r, 0)),
            pl.BlockSpec((BT, 128), lambda r: (r, 0)),
        ],
        out_shape=jax.ShapeDtypeStruct((T, D), jnp.float32),
        out_specs=pl.BlockSpec((BT, D), lambda r: (r, 0)),
    )(y01, y01, w0b, w1b)


def kernel(hidden_states, topk_indices, topk_weights, gate_up_proj, down_proj):
    flat_e = topk_indices.reshape(-1).astype(jnp.int32)
    order = jnp.argsort(flat_e)
    counts = jnp.zeros((E,), jnp.int32).at[flat_e].add(1)
    off = jnp.concatenate([jnp.zeros((1,), jnp.int32),
                           jnp.cumsum(counts)[:-1].astype(jnp.int32)])
    padded = ((counts + BM - 1) // BM) * BM
    starts = jnp.concatenate([jnp.zeros((1,), jnp.int32),
                              jnp.cumsum(padded)[:-1].astype(jnp.int32)])

    block_start = jnp.arange(NB, dtype=jnp.int32) * BM
    be = (jnp.searchsorted(starts, block_start, side="right")
          .astype(jnp.int32) - 1)

    p = jnp.arange(NP, dtype=jnp.int32)
    e_p = be[p // BM]
    r = p - starts[e_p]
    valid = r < counts[e_p]
    j = jnp.where(valid, off[e_p] + r, 0)
    fi = order[j]
    src_tok = jnp.where(valid, fi // K, 0).astype(jnp.int32)

    inv = jnp.zeros((NTOT,), jnp.int32).at[order].set(
        jnp.arange(NTOT, dtype=jnp.int32))
    pos = starts[flat_e] + (inv - off[flat_e])
    pos2 = pos.reshape(T, K)
    pos_cat = jnp.concatenate([pos2[:, 0], pos2[:, 1]])

    w0b = jnp.broadcast_to(topk_weights[:, 0:1], (T, 128))
    w1b = jnp.broadcast_to(topk_weights[:, 1:2], (T, 128))

    gu_b = gate_up_proj.astype(jnp.bfloat16)
    dn_b = down_proj.astype(jnp.bfloat16)

    x_pad = _sc_gather(hidden_states, src_tok, NP)
    h_pad = _stage1(be, x_pad, gu_b)
    y_pad = _stage2(be, h_pad, dn_b)
    y01 = _sc_gather(y_pad, pos_cat, NTOT)
    return _combine(y01, w0b, w1b)


# speedup vs baseline: 1.2405x; 1.2405x over previous
import functools

import jax
import jax.numpy as jnp
from jax import lax
from jax.experimental import pallas as pl
from jax.experimental.pallas import tpu as pltpu
from jax.experimental.pallas import tpu_sc as plsc

E = 8
K = 2
D = 2048
F = 2048
T = 2048
NTOT = T * K

BM = 128
NP = NTOT + E * BM
NB = NP // BM
BF = 1024
NF1 = F // BF
NF2 = D // BF

NC, NS = 2, 16
NW = NC * NS
CH = 16

BT = 256


def _sc_gather(table, idx, n_rows):
    rows_per_w = n_rows // NW
    chunks = rows_per_w // CH
    mesh = plsc.VectorSubcoreMesh(core_axis_name="c", subcore_axis_name="s")

    @functools.partial(
        pl.kernel,
        mesh=mesh,
        out_type=jax.ShapeDtypeStruct((n_rows, D), jnp.float32),
        scratch_types=[
            pltpu.VMEM((rows_per_w,), jnp.int32),
            pltpu.VMEM((CH, D), jnp.float32),
            pltpu.VMEM((CH, D), jnp.float32),
            pltpu.SemaphoreType.DMA,
            pltpu.SemaphoreType.DMA,
            pltpu.SemaphoreType.DMA,
            pltpu.SemaphoreType.DMA,
        ],
    )
    def gk(table_hbm, idx_hbm, out_hbm, idx_v, buf0, buf1, g0, g1, w0, w1):
        wid = lax.axis_index("s") * NC + lax.axis_index("c")
        base = wid * rows_per_w
        pltpu.sync_copy(idx_hbm.at[pl.ds(base, rows_per_w)], idx_v)
        bufs = (buf0, buf1)
        gsems = (g0, g1)
        wsems = (w0, w1)

        def gather(t):
            return pltpu.async_copy(
                table_hbm.at[idx_v.at[pl.ds(t * CH, CH)]],
                bufs[t % 2], gsems[t % 2])

        def writeback(t):
            return pltpu.async_copy(
                bufs[t % 2], out_hbm.at[pl.ds(base + t * CH, CH)],
                wsems[t % 2])

        pending_g = gather(0)
        pending_w = [None, None]
        for t in range(chunks):
            pending_g.wait()
            if t + 1 < chunks:
                if pending_w[(t + 1) % 2] is not None:
                    pending_w[(t + 1) % 2].wait()
                    pending_w[(t + 1) % 2] = None
                pending_g = gather(t + 1)
            pending_w[t % 2] = writeback(t)
        for w in pending_w:
            if w is not None:
                w.wait()

    return gk(table, idx)


def _stage1(be, x_pad, gate_up):
    def body(be_ref, x_ref, wg_ref, wu_ref, h_ref):
        del be_ref
        x = x_ref[...]
        g = jnp.dot(x, wg_ref[0], preferred_element_type=jnp.float32)
        u = jnp.dot(x, wu_ref[0], preferred_element_type=jnp.float32)
        h_ref[...] = ((g * jax.nn.sigmoid(g)) * u).astype(jnp.bfloat16)

    grid_spec = pltpu.PrefetchScalarGridSpec(
        num_scalar_prefetch=1,
        grid=(NF1, NB),
        in_specs=[
            pl.BlockSpec((BM, D), lambda f, r, be: (r, 0)),
            pl.BlockSpec((1, D, BF), lambda f, r, be: (be[r], 0, f)),
            pl.BlockSpec((1, D, BF), lambda f, r, be: (be[r], 0, f + NF1)),
        ],
        out_specs=pl.BlockSpec((BM, BF), lambda f, r, be: (r, f)),
    )
    return pl.pallas_call(
        body,
        grid_spec=grid_spec,
        out_shape=jax.ShapeDtypeStruct((NP, F), jnp.bfloat16),
    )(be, x_pad, gate_up, gate_up)


def _stage2(be, h_pad, down_proj):
    def body(be_ref, h_ref, wd_ref, y_ref):
        del be_ref
        y_ref[...] = jnp.dot(h_ref[...].astype(jnp.float32), wd_ref[0],
                             preferred_element_type=jnp.float32)

    grid_spec = pltpu.PrefetchScalarGridSpec(
        num_scalar_prefetch=1,
        grid=(NF2, NB),
        in_specs=[
            pl.BlockSpec((BM, F), lambda f, r, be: (r, 0)),
            pl.BlockSpec((1, F, BF), lambda f, r, be: (be[r], 0, f)),
        ],
        out_specs=pl.BlockSpec((BM, BF), lambda f, r, be: (r, f)),
    )
    return pl.pallas_call(
        body,
        grid_spec=grid_spec,
        out_shape=jax.ShapeDtypeStruct((NP, D), jnp.float32),
    )(be, h_pad, down_proj)


def _combine(y01, w0b, w1b):
    nb = T // BT

    def body(ya_ref, yb_ref, wa_ref, wb_ref, o_ref):
        o_ref[...] = (wa_ref[:, :1] * ya_ref[...]
                      + wb_ref[:, :1] * yb_ref[...])

    return pl.pallas_call(
        body,
        grid=(nb,),
        in_specs=[
            pl.BlockSpec((BT, D), lambda r: (r, 0)),
            pl.BlockSpec((BT, D), lambda r: (r + nb, 0)),
            pl.BlockSpec((BT, 128), lambda r: (r, 0)),
            pl.BlockSpec((BT, 128), lambda r: (r, 0)),
        ],
        out_shape=jax.ShapeDtypeStruct((T, D), jnp.float32),
        out_specs=pl.BlockSpec((BT, D), lambda r: (r, 0)),
    )(y01, y01, w0b, w1b)


def kernel(hidden_states, topk_indices, topk_weights, gate_up_proj, down_proj):
    flat_e = topk_indices.reshape(-1).astype(jnp.int32)
    order = jnp.argsort(flat_e)
    counts = jnp.zeros((E,), jnp.int32).at[flat_e].add(1)
    off = jnp.concatenate([jnp.zeros((1,), jnp.int32),
                           jnp.cumsum(counts)[:-1].astype(jnp.int32)])
    padded = ((counts + BM - 1) // BM) * BM
    starts = jnp.concatenate([jnp.zeros((1,), jnp.int32),
                              jnp.cumsum(padded)[:-1].astype(jnp.int32)])

    block_start = jnp.arange(NB, dtype=jnp.int32) * BM
    be = (jnp.searchsorted(starts, block_start, side="right")
          .astype(jnp.int32) - 1)

    p = jnp.arange(NP, dtype=jnp.int32)
    e_p = be[p // BM]
    r = p - starts[e_p]
    valid = r < counts[e_p]
    j = jnp.where(valid, off[e_p] + r, 0)
    fi = order[j]
    src_tok = jnp.where(valid, fi // K, 0).astype(jnp.int32)

    inv = jnp.zeros((NTOT,), jnp.int32).at[order].set(
        jnp.arange(NTOT, dtype=jnp.int32))
    pos = starts[flat_e] + (inv - off[flat_e])
    pos2 = pos.reshape(T, K)
    pos_cat = jnp.concatenate([pos2[:, 0], pos2[:, 1]])

    w0b = jnp.broadcast_to(topk_weights[:, 0:1], (T, 128))
    w1b = jnp.broadcast_to(topk_weights[:, 1:2], (T, 128))

    x_pad = _sc_gather(hidden_states, src_tok, NP)
    h_pad = _stage1(be, x_pad, gate_up_proj)
    y_pad = _stage2(be, h_pad, down_proj)
    y01 = _sc_gather(y_pad, pos_cat, NTOT)
    return _combine(y01, w0b, w1b)


# speedup vs baseline: 1.6039x; 1.2929x over previous
import functools

import jax
import jax.numpy as jnp
from jax import lax
from jax.experimental import pallas as pl
from jax.experimental.pallas import tpu as pltpu
from jax.experimental.pallas import tpu_sc as plsc

E = 8
K = 2
D = 2048
F = 2048
T = 2048
NTOT = T * K

BM = 128
NP = NTOT + E * BM
NB = NP // BM
BF = 1024
NF1 = F // BF
NF2 = D // BF

NC, NS = 2, 16
NW = NC * NS
CH = 16

BT = 256

TOK_PER_W = T // NW
DCH = 16
DCHUNKS = TOK_PER_W // DCH


def _sc_gather(table, idx, n_rows):
    rows_per_w = n_rows // NW
    chunks = rows_per_w // CH
    mesh = plsc.VectorSubcoreMesh(core_axis_name="c", subcore_axis_name="s")

    @functools.partial(
        pl.kernel,
        mesh=mesh,
        out_type=jax.ShapeDtypeStruct((n_rows, D), jnp.float32),
        scratch_types=[
            pltpu.VMEM((rows_per_w,), jnp.int32),
            pltpu.VMEM((CH, D), jnp.float32),
            pltpu.VMEM((CH, D), jnp.float32),
            pltpu.SemaphoreType.DMA,
            pltpu.SemaphoreType.DMA,
            pltpu.SemaphoreType.DMA,
            pltpu.SemaphoreType.DMA,
        ],
    )
    def gk(table_hbm, idx_hbm, out_hbm, idx_v, buf0, buf1, g0, g1, w0, w1):
        wid = lax.axis_index("s") * NC + lax.axis_index("c")
        base = wid * rows_per_w
        pltpu.sync_copy(idx_hbm.at[pl.ds(base, rows_per_w)], idx_v)
        bufs = (buf0, buf1)
        gsems = (g0, g1)
        wsems = (w0, w1)

        def gather(t):
            return pltpu.async_copy(
                table_hbm.at[idx_v.at[pl.ds(t * CH, CH)]],
                bufs[t % 2], gsems[t % 2])

        def writeback(t):
            return pltpu.async_copy(
                bufs[t % 2], out_hbm.at[pl.ds(base + t * CH, CH)],
                wsems[t % 2])

        pending_g = gather(0)
        pending_w = [None, None]
        for t in range(chunks):
            pending_g.wait()
            if t + 1 < chunks:
                if pending_w[(t + 1) % 2] is not None:
                    pending_w[(t + 1) % 2].wait()
                    pending_w[(t + 1) % 2] = None
                pending_g = gather(t + 1)
            pending_w[t % 2] = writeback(t)
        for w in pending_w:
            if w is not None:
                w.wait()

    return gk(table, idx)


def _sc_dispatch(hidden, idx0, idx1):
    mesh = plsc.VectorSubcoreMesh(core_axis_name="c", subcore_axis_name="s")

    @functools.partial(
        pl.kernel,
        mesh=mesh,
        out_type=jax.ShapeDtypeStruct((NP, D), jnp.float32),
        scratch_types=[
            pltpu.VMEM((DCHUNKS, DCH), jnp.int32),
            pltpu.VMEM((DCHUNKS, DCH), jnp.int32),
            pltpu.VMEM((DCH, D), jnp.float32),
            pltpu.VMEM((DCH, D), jnp.float32),
            pltpu.SemaphoreType.DMA,
            pltpu.SemaphoreType.DMA,
            pltpu.SemaphoreType.DMA,
            pltpu.SemaphoreType.DMA,
        ],
    )
    def dk(hid_hbm, i0_hbm, i1_hbm, out_hbm, i0_v, i1_v, buf0, buf1,
           g0, g1, s0, s1):
        wid = lax.axis_index("s") * NC + lax.axis_index("c")
        pltpu.sync_copy(i0_hbm.at[wid], i0_v)
        pltpu.sync_copy(i1_hbm.at[wid], i1_v)
        bufs = (buf0, buf1)
        gsems = (g0, g1)
        ssems = (s0, s1)
        base = wid * TOK_PER_W

        def rd(t):
            return pltpu.async_copy(
                hid_hbm.at[pl.ds(base + t * DCH, DCH)], bufs[t % 2],
                gsems[t % 2])

        def wr(t):
            b = bufs[t % 2]
            return (pltpu.async_copy(b, out_hbm.at[i0_v.at[t]], ssems[t % 2]),
                    pltpu.async_copy(b, out_hbm.at[i1_v.at[t]], ssems[t % 2]))

        pending_g = rd(0)
        pending_w = [None, None]
        for t in range(DCHUNKS):
            pending_g.wait()
            if t + 1 < DCHUNKS:
                if pending_w[(t + 1) % 2] is not None:
                    for c in pending_w[(t + 1) % 2]:
                        c.wait()
                    pending_w[(t + 1) % 2] = None
                pending_g = rd(t + 1)
            pending_w[t % 2] = wr(t)
        for p in pending_w:
            if p is not None:
                for c in p:
                    c.wait()

    return dk(hidden, idx0, idx1)


def _stage1(be, x_pad, gate_up):
    def body(be_ref, x_ref, wg_ref, wu_ref, h_ref):
        del be_ref
        x = x_ref[...]
        g = jnp.dot(x, wg_ref[0], preferred_element_type=jnp.float32)
        u = jnp.dot(x, wu_ref[0], preferred_element_type=jnp.float32)
        h_ref[...] = ((g * jax.nn.sigmoid(g)) * u).astype(jnp.bfloat16)

    grid_spec = pltpu.PrefetchScalarGridSpec(
        num_scalar_prefetch=1,
        grid=(NF1, NB),
        in_specs=[
            pl.BlockSpec((BM, D), lambda f, r, be: (r, 0)),
            pl.BlockSpec((1, D, BF), lambda f, r, be: (be[r], 0, f)),
            pl.BlockSpec((1, D, BF), lambda f, r, be: (be[r], 0, f + NF1)),
        ],
        out_specs=pl.BlockSpec((BM, BF), lambda f, r, be: (r, f)),
    )
    return pl.pallas_call(
        body,
        grid_spec=grid_spec,
        out_shape=jax.ShapeDtypeStruct((NP, F), jnp.bfloat16),
    )(be, x_pad, gate_up, gate_up)


def _stage2(be, h_pad, down_proj):
    def body(be_ref, h_ref, wd_ref, y_ref):
        del be_ref
        y_ref[...] = jnp.dot(h_ref[...].astype(jnp.float32), wd_ref[0],
                             preferred_element_type=jnp.float32)

    grid_spec = pltpu.PrefetchScalarGridSpec(
        num_scalar_prefetch=1,
        grid=(NF2, NB),
        in_specs=[
            pl.BlockSpec((BM, F), lambda f, r, be: (r, 0)),
            pl.BlockSpec((1, F, BF), lambda f, r, be: (be[r], 0, f)),
        ],
        out_specs=pl.BlockSpec((BM, BF), lambda f, r, be: (r, f)),
    )
    return pl.pallas_call(
        body,
        grid_spec=grid_spec,
        out_shape=jax.ShapeDtypeStruct((NP, D), jnp.float32),
    )(be, h_pad, down_proj)


def _combine(y01, w0b, w1b):
    nb = T // BT

    def body(ya_ref, yb_ref, wa_ref, wb_ref, o_ref):
        o_ref[...] = (wa_ref[:, :1] * ya_ref[...]
                      + wb_ref[:, :1] * yb_ref[...])

    return pl.pallas_call(
        body,
        grid=(nb,),
        in_specs=[
            pl.BlockSpec((BT, D), lambda r: (r, 0)),
            pl.BlockSpec((BT, D), lambda r: (r + nb, 0)),
            pl.BlockSpec((BT, 128), lambda r: (r, 0)),
            pl.BlockSpec((BT, 128), lambda r: (r, 0)),
        ],
        out_shape=jax.ShapeDtypeStruct((T, D), jnp.float32),
        out_specs=pl.BlockSpec((BT, D), lambda r: (r, 0)),
    )(y01, y01, w0b, w1b)


def kernel(hidden_states, topk_indices, topk_weights, gate_up_proj, down_proj):
    flat_e = topk_indices.reshape(-1).astype(jnp.int32)
    order = jnp.argsort(flat_e)
    counts = jnp.zeros((E,), jnp.int32).at[flat_e].add(1)
    off = jnp.concatenate([jnp.zeros((1,), jnp.int32),
                           jnp.cumsum(counts)[:-1].astype(jnp.int32)])
    padded = ((counts + BM - 1) // BM) * BM
    starts = jnp.concatenate([jnp.zeros((1,), jnp.int32),
                              jnp.cumsum(padded)[:-1].astype(jnp.int32)])

    block_start = jnp.arange(NB, dtype=jnp.int32) * BM
    be = (jnp.searchsorted(starts, block_start, side="right")
          .astype(jnp.int32) - 1)

    inv = jnp.zeros((NTOT,), jnp.int32).at[order].set(
        jnp.arange(NTOT, dtype=jnp.int32))
    pos = starts[flat_e] + (inv - off[flat_e])
    pos2 = pos.reshape(T, K)
    pos_cat = jnp.concatenate([pos2[:, 0], pos2[:, 1]])

    idx0 = pos2[:, 0].reshape(NW, DCHUNKS, DCH).astype(jnp.int32)
    idx1 = pos2[:, 1].reshape(NW, DCHUNKS, DCH).astype(jnp.int32)

    w0b = jnp.broadcast_to(topk_weights[:, 0:1], (T, 128))
    w1b = jnp.broadcast_to(topk_weights[:, 1:2], (T, 128))

    x_pad = _sc_dispatch(hidden_states, idx0, idx1)
    h_pad = _stage1(be, x_pad, gate_up_proj)
    y_pad = _stage2(be, h_pad, down_proj)
    y01 = _sc_gather(y_pad, pos_cat, NTOT)
    return _combine(y01, w0b, w1b)


# speedup vs baseline: 1.7060x; 1.0637x over previous
import functools

import jax
import jax.numpy as jnp
from jax import lax
from jax.experimental import pallas as pl
from jax.experimental.pallas import tpu as pltpu
from jax.experimental.pallas import tpu_sc as plsc

E = 8
K = 2
D = 2048
F = 2048
T = 2048
NTOT = T * K

BM = 128
NP = NTOT + E * BM
NB = NP // BM
BF = 1024
NF1 = F // BF
BF2 = 2048
NF2 = D // BF2

NC, NS = 2, 16
NW = NC * NS
CH = 16

BT = 256

TOK_PER_W = T // NW
DCH = 16
DCHUNKS = TOK_PER_W // DCH


def _sc_gather(table, idx, n_rows):
    rows_per_w = n_rows // NW
    chunks = rows_per_w // CH
    dt = table.dtype
    mesh = plsc.VectorSubcoreMesh(core_axis_name="c", subcore_axis_name="s")

    @functools.partial(
        pl.kernel,
        mesh=mesh,
        out_type=jax.ShapeDtypeStruct((n_rows, D), dt),
        scratch_types=[
            pltpu.VMEM((rows_per_w,), jnp.int32),
            pltpu.VMEM((CH, D), dt),
            pltpu.VMEM((CH, D), dt),
            pltpu.SemaphoreType.DMA,
            pltpu.SemaphoreType.DMA,
            pltpu.SemaphoreType.DMA,
            pltpu.SemaphoreType.DMA,
        ],
    )
    def gk(table_hbm, idx_hbm, out_hbm, idx_v, buf0, buf1, g0, g1, w0, w1):
        wid = lax.axis_index("s") * NC + lax.axis_index("c")
        base = wid * rows_per_w
        pltpu.sync_copy(idx_hbm.at[pl.ds(base, rows_per_w)], idx_v)
        bufs = (buf0, buf1)
        gsems = (g0, g1)
        wsems = (w0, w1)

        def gather(t):
            return pltpu.async_copy(
                table_hbm.at[idx_v.at[pl.ds(t * CH, CH)]],
                bufs[t % 2], gsems[t % 2])

        def writeback(t):
            return pltpu.async_copy(
                bufs[t % 2], out_hbm.at[pl.ds(base + t * CH, CH)],
                wsems[t % 2])

        pending_g = gather(0)
        pending_w = [None, None]
        for t in range(chunks):
            pending_g.wait()
            if t + 1 < chunks:
                if pending_w[(t + 1) % 2] is not None:
                    pending_w[(t + 1) % 2].wait()
                    pending_w[(t + 1) % 2] = None
                pending_g = gather(t + 1)
            pending_w[t % 2] = writeback(t)
        for w in pending_w:
            if w is not None:
                w.wait()

    return gk(table, idx)


def _sc_dispatch(hidden, idx0, idx1):
    dt = hidden.dtype
    mesh = plsc.VectorSubcoreMesh(core_axis_name="c", subcore_axis_name="s")

    @functools.partial(
        pl.kernel,
        mesh=mesh,
        out_type=jax.ShapeDtypeStruct((NP, D), dt),
        scratch_types=[
            pltpu.VMEM((DCHUNKS, DCH), jnp.int32),
            pltpu.VMEM((DCHUNKS, DCH), jnp.int32),
            pltpu.VMEM((DCH, D), dt),
            pltpu.VMEM((DCH, D), dt),
            pltpu.SemaphoreType.DMA,
            pltpu.SemaphoreType.DMA,
            pltpu.SemaphoreType.DMA,
            pltpu.SemaphoreType.DMA,
        ],
    )
    def dk(hid_hbm, i0_hbm, i1_hbm, out_hbm, i0_v, i1_v, buf0, buf1,
           g0, g1, s0, s1):
        wid = lax.axis_index("s") * NC + lax.axis_index("c")
        pltpu.sync_copy(i0_hbm.at[wid], i0_v)
        pltpu.sync_copy(i1_hbm.at[wid], i1_v)
        bufs = (buf0, buf1)
        gsems = (g0, g1)
        ssems = (s0, s1)
        base = wid * TOK_PER_W

        def rd(t):
            return pltpu.async_copy(
                hid_hbm.at[pl.ds(base + t * DCH, DCH)], bufs[t % 2],
                gsems[t % 2])

        def wr(t):
            b = bufs[t % 2]
            return (pltpu.async_copy(b, out_hbm.at[i0_v.at[t]], ssems[t % 2]),
                    pltpu.async_copy(b, out_hbm.at[i1_v.at[t]], ssems[t % 2]))

        pending_g = rd(0)
        pending_w = [None, None]
        for t in range(DCHUNKS):
            pending_g.wait()
            if t + 1 < DCHUNKS:
                if pending_w[(t + 1) % 2] is not None:
                    for c in pending_w[(t + 1) % 2]:
                        c.wait()
                    pending_w[(t + 1) % 2] = None
                pending_g = rd(t + 1)
            pending_w[t % 2] = wr(t)
        for p in pending_w:
            if p is not None:
                for c in p:
                    c.wait()

    return dk(hidden, idx0, idx1)


def _stage1(be, x_pad, gate_up):
    def body(be_ref, x_ref, wg_ref, wu_ref, h_ref):
        del be_ref
        x = x_ref[...].astype(jnp.float32)
        g = jnp.dot(x, wg_ref[0], preferred_element_type=jnp.float32)
        u = jnp.dot(x, wu_ref[0], preferred_element_type=jnp.float32)
        h_ref[...] = ((g * jax.nn.sigmoid(g)) * u).astype(jnp.bfloat16)

    grid_spec = pltpu.PrefetchScalarGridSpec(
        num_scalar_prefetch=1,
        grid=(NF1, NB),
        in_specs=[
            pl.BlockSpec((BM, D), lambda f, r, be: (r, 0)),
            pl.BlockSpec((1, D, BF), lambda f, r, be: (be[r], 0, f)),
            pl.BlockSpec((1, D, BF), lambda f, r, be: (be[r], 0, f + NF1)),
        ],
        out_specs=pl.BlockSpec((BM, BF), lambda f, r, be: (r, f)),
    )
    return pl.pallas_call(
        body,
        grid_spec=grid_spec,
        out_shape=jax.ShapeDtypeStruct((NP, F), jnp.bfloat16),
    )(be, x_pad, gate_up, gate_up)


def _stage2(be, h_pad, down_proj):
    def body(be_ref, h_ref, wd_ref, y_ref):
        del be_ref
        y_ref[...] = jnp.dot(h_ref[...].astype(jnp.float32), wd_ref[0],
                             preferred_element_type=jnp.float32)

    grid_spec = pltpu.PrefetchScalarGridSpec(
        num_scalar_prefetch=1,
        grid=(NF2, NB),
        in_specs=[
            pl.BlockSpec((BM, F), lambda f, r, be: (r, 0)),
            pl.BlockSpec((1, F, BF2), lambda f, r, be: (be[r], 0, f)),
        ],
        out_specs=pl.BlockSpec((BM, BF2), lambda f, r, be: (r, f)),
    )
    return pl.pallas_call(
        body,
        grid_spec=grid_spec,
        out_shape=jax.ShapeDtypeStruct((NP, D), jnp.float32),
    )(be, h_pad, down_proj)


def _combine(y01, w0b, w1b):
    nb = T // BT

    def body(ya_ref, yb_ref, wa_ref, wb_ref, o_ref):
        o_ref[...] = (wa_ref[:, :1] * ya_ref[...]
                      + wb_ref[:, :1] * yb_ref[...])

    return pl.pallas_call(
        body,
        grid=(nb,),
        in_specs=[
            pl.BlockSpec((BT, D), lambda r: (r, 0)),
            pl.BlockSpec((BT, D), lambda r: (r + nb, 0)),
            pl.BlockSpec((BT, 128), lambda r: (r, 0)),
            pl.BlockSpec((BT, 128), lambda r: (r, 0)),
        ],
        out_shape=jax.ShapeDtypeStruct((T, D), jnp.float32),
        out_specs=pl.BlockSpec((BT, D), lambda r: (r, 0)),
    )(y01, y01, w0b, w1b)


def kernel(hidden_states, topk_indices, topk_weights, gate_up_proj, down_proj):
    flat_e = topk_indices.reshape(-1).astype(jnp.int32)
    order = jnp.argsort(flat_e)
    counts = jnp.zeros((E,), jnp.int32).at[flat_e].add(1)
    off = jnp.concatenate([jnp.zeros((1,), jnp.int32),
                           jnp.cumsum(counts)[:-1].astype(jnp.int32)])
    padded = ((counts + BM - 1) // BM) * BM
    starts = jnp.concatenate([jnp.zeros((1,), jnp.int32),
                              jnp.cumsum(padded)[:-1].astype(jnp.int32)])

    block_start = jnp.arange(NB, dtype=jnp.int32) * BM
    be = (jnp.searchsorted(starts, block_start, side="right")
          .astype(jnp.int32) - 1)

    inv = jnp.zeros((NTOT,), jnp.int32).at[order].set(
        jnp.arange(NTOT, dtype=jnp.int32))
    pos = starts[flat_e] + (inv - off[flat_e])
    pos2 = pos.reshape(T, K)
    pos_cat = jnp.concatenate([pos2[:, 0], pos2[:, 1]])

    idx0 = pos2[:, 0].reshape(NW, DCHUNKS, DCH).astype(jnp.int32)
    idx1 = pos2[:, 1].reshape(NW, DCHUNKS, DCH).astype(jnp.int32)

    w0b = jnp.broadcast_to(topk_weights[:, 0:1], (T, 128))
    w1b = jnp.broadcast_to(topk_weights[:, 1:2], (T, 128))

    x_pad = _sc_dispatch(hidden_states, idx0, idx1)
    h_pad = _stage1(be, x_pad, gate_up_proj)
    y_pad = _stage2(be, h_pad, down_proj)
    y01 = _sc_gather(y_pad, pos_cat, NTOT)
    return _combine(y01, w0b, w1b)
